# baseline (speedup 1.0000x reference)
import functools

import jax
import jax.numpy as jnp
from jax.experimental import pallas as pl
from jax.experimental.pallas import tpu as pltpu

_BN = 2048
_LANES = 256


def _body(x_ref, w_ref, eo_ref, ft_ref,
          act_ref, val_ref, lp_ref, no_ref, mv_ref, mlp_ref, tp_ref,
          *, ea, e, a):
    x = x_ref[...]
    w = w_ref[...]
    acc = jnp.dot(x, w, preferred_element_type=jnp.float32)
    bn = acc.shape[0]
    eo = eo_ref[0]
    ft = ft_ref[0]
    neg = jnp.float32(-1e30)

    acc_p = acc[:, :ea].T
    acc_h = acc[:, ea:ea + 2 * e + 1 + e].T

    meta = acc_h[0:e]
    srow = jax.lax.broadcasted_iota(jnp.int32, (e, bn), 0)
    mmax = jnp.max(meta, axis=0, keepdims=True)
    marg = jnp.min(jnp.where(meta == mmax, srow, e), axis=0, keepdims=True)
    msum = jnp.sum(jnp.exp(meta - mmax), axis=0, keepdims=True)
    meta_logp = -jnp.log(msum)
    meta_val = acc_h[e:e + 1]

    t8 = acc_h[e + 1:2 * e + 1]
    tlog = jnp.sum(jnp.where(srow == eo, t8, 0.0), axis=0, keepdims=True)
    tprob = jax.nn.sigmoid(tlog)
    req = (tprob > 0.5) | (ft > 0)
    newopt = jnp.where(req, marg, eo)
    tout = jnp.where(ft > 0, jnp.float32(0.0), tprob)

    acc3 = acc_p.reshape(e, a, bn)
    erow = jax.lax.broadcasted_iota(jnp.int32, (e, a, bn), 0)
    sel = jnp.sum(jnp.where(erow == newopt[None], acc3, 0.0), axis=0)
    arow = jax.lax.broadcasted_iota(jnp.int32, (a, bn), 0)
    smax = jnp.max(sel, axis=0, keepdims=True)
    sarg = jnp.min(jnp.where(sel == smax, arow, a), axis=0, keepdims=True)
    ssum = jnp.sum(jnp.exp(sel - smax), axis=0, keepdims=True)
    lp = -jnp.log(ssum)
    v8 = acc_h[2 * e + 1:3 * e + 1]
    val = jnp.sum(jnp.where(srow == newopt, v8, 0.0), axis=0, keepdims=True)

    act_ref[0] = sarg
    val_ref[0] = val
    lp_ref[0] = lp
    no_ref[0] = newopt
    mv_ref[0] = meta_val
    mlp_ref[0] = meta_logp
    tp_ref[0] = tout


def kernel(observation, first_transition, executing_option, Wm, Wmv, Wt, Wp, Wv):
    n, d = observation.shape
    e = Wm.shape[1]
    a = Wp.shape[2]
    ea = e * a
    ncols = ea + 2 * e + 1 + e
    nblk = n // _BN
    wp_flat = jnp.transpose(Wp, (1, 0, 2)).reshape(d, ea)
    w_all = jnp.concatenate(
        [wp_flat, Wm, Wmv, Wt, Wv[..., 0].T,
         jnp.zeros((d, _LANES - ncols), jnp.float32)], axis=1)
    eo3 = executing_option.astype(jnp.int32).reshape(nblk, 1, _BN)
    ft3 = first_transition.astype(jnp.int32).reshape(nblk, 1, _BN)

    row_spec = pl.BlockSpec((1, 1, _BN), lambda i: (i, 0, 0))
    o_f32 = jax.ShapeDtypeStruct((nblk, 1, _BN), jnp.float32)
    o_i32 = jax.ShapeDtypeStruct((nblk, 1, _BN), jnp.int32)
    outs = pl.pallas_call(
        functools.partial(_body, ea=ea, e=e, a=a),
        grid=(nblk,),
        in_specs=[
            pl.BlockSpec((_BN, d), lambda i: (i, 0)),
            pl.BlockSpec((d, _LANES), lambda i: (0, 0)),
            row_spec,
            row_spec,
        ],
        out_specs=[row_spec] * 7,
        out_shape=[o_i32, o_f32, o_f32, o_i32, o_f32, o_f32, o_f32],
        compiler_params=pltpu.CompilerParams(
            dimension_semantics=("arbitrary",)),
    )(observation, w_all, eo3, ft3)
    return tuple(o.reshape(n) for o in outs)


# speedup vs baseline: 1.0502x; 1.0502x over previous
import functools

import jax
import jax.numpy as jnp
from jax.experimental import pallas as pl
from jax.experimental.pallas import tpu as pltpu

_BN = 1024
_LANES = 256


def _body(x1_ref, x2_ref, w_ref, eo_ref, ft_ref,
          act_ref, val_ref, lp_ref, no_ref, mv_ref, mlp_ref, tp_ref,
          *, ea, e, a):
    w = w_ref[...]
    dh = x1_ref.shape[1]
    acc = (jnp.dot(x1_ref[...], w[:dh], preferred_element_type=jnp.float32)
           + jnp.dot(x2_ref[...], w[dh:], preferred_element_type=jnp.float32))
    bn = acc.shape[0]
    eo = eo_ref[0]
    ft = ft_ref[0]
    neg = jnp.float32(-1e30)

    acc_p = acc[:, :ea].T
    acc_h = acc[:, ea:ea + 2 * e + 1 + e].T

    meta = acc_h[0:e]
    srow = jax.lax.broadcasted_iota(jnp.int32, (e, bn), 0)
    mmax = jnp.max(meta, axis=0, keepdims=True)
    marg = jnp.min(jnp.where(meta == mmax, srow, e), axis=0, keepdims=True)
    msum = jnp.sum(jnp.exp(meta - mmax), axis=0, keepdims=True)
    meta_logp = -jnp.log(msum)
    meta_val = acc_h[e:e + 1]

    t8 = acc_h[e + 1:2 * e + 1]
    tlog = jnp.sum(jnp.where(srow == eo, t8, 0.0), axis=0, keepdims=True)
    tprob = jax.nn.sigmoid(tlog)
    req = (tprob > 0.5) | (ft > 0)
    newopt = jnp.where(req, marg, eo)
    tout = jnp.where(ft > 0, jnp.float32(0.0), tprob)

    acc3 = acc_p.reshape(e, a, bn)
    erow = jax.lax.broadcasted_iota(jnp.int32, (e, a, bn), 0)
    sel = jnp.sum(jnp.where(erow == newopt[None], acc3, 0.0), axis=0)
    arow = jax.lax.broadcasted_iota(jnp.int32, (a, bn), 0)
    smax = jnp.max(sel, axis=0, keepdims=True)
    sarg = jnp.min(jnp.where(sel == smax, arow, a), axis=0, keepdims=True)
    ssum = jnp.sum(jnp.exp(sel - smax), axis=0, keepdims=True)
    lp = -jnp.log(ssum)
    v8 = acc_h[2 * e + 1:3 * e + 1]
    val = jnp.sum(jnp.where(srow == newopt, v8, 0.0), axis=0, keepdims=True)

    act_ref[0] = sarg
    val_ref[0] = val
    lp_ref[0] = lp
    no_ref[0] = newopt
    mv_ref[0] = meta_val
    mlp_ref[0] = meta_logp
    tp_ref[0] = tout


def kernel(observation, first_transition, executing_option, Wm, Wmv, Wt, Wp, Wv):
    n, d = observation.shape
    e = Wm.shape[1]
    a = Wp.shape[2]
    ea = e * a
    ncols = ea + 2 * e + 1 + e
    nblk = n // _BN
    wp_flat = jnp.transpose(Wp, (1, 0, 2)).reshape(d, ea)
    w_all = jnp.concatenate(
        [wp_flat, Wm, Wmv, Wt, Wv[..., 0].T,
         jnp.zeros((d, _LANES - ncols), jnp.float32)], axis=1)
    eo3 = executing_option.astype(jnp.int32).reshape(nblk, 1, _BN)
    ft3 = first_transition.astype(jnp.int32).reshape(nblk, 1, _BN)

    row_spec = pl.BlockSpec((1, 1, _BN), lambda i: (i, 0, 0))
    o_f32 = jax.ShapeDtypeStruct((nblk, 1, _BN), jnp.float32)
    o_i32 = jax.ShapeDtypeStruct((nblk, 1, _BN), jnp.int32)
    outs = pl.pallas_call(
        functools.partial(_body, ea=ea, e=e, a=a),
        grid=(nblk,),
        in_specs=[
            pl.BlockSpec((_BN, d // 2), lambda i: (i, 0)),
            pl.BlockSpec((_BN, d // 2), lambda i: (i, 1)),
            pl.BlockSpec((d, _LANES), lambda i: (0, 0)),
            row_spec,
            row_spec,
        ],
        out_specs=[row_spec] * 7,
        out_shape=[o_i32, o_f32, o_f32, o_i32, o_f32, o_f32, o_f32],
        compiler_params=pltpu.CompilerParams(
            dimension_semantics=("arbitrary",)),
    )(observation, observation, w_all, eo3, ft3)
    return tuple(o.reshape(n) for o in outs)
